# baseline (speedup 1.0000x reference)
import functools

import jax
import jax.numpy as jnp
from jax import lax
from jax.experimental import pallas as pl
from jax.experimental.pallas import tpu as pltpu
from jax.experimental.pallas import tpu_sc as plsc

B, S, G = 1024, 50, 8
D = 64
VOCAB = 1000000
N = B * S * G
BS = B * S
NC, NS = 2, 16
NW = NC * NS
PER_W = N // NW
ROWS_W = BS // NW
CHUNK = 128
SLAB = 1280
NSLAB = PER_W // SLAB
CH_PER_SLAB = SLAB // CHUNK
OUT_SLAB = SLAB // G
LANES = 16

_mesh = plsc.VectorSubcoreMesh(core_axis_name="c", subcore_axis_name="s")

HV = VOCAB // 2
PKB = 128
PGRID = (HV + PKB - 1) // PKB
RSHIFT = HV // PKB


def _pack_body(l_ref, ra_ref, rb_ref, o_ref):
    right = jnp.concatenate([ra_ref[:, 32:128], rb_ref[:, 0:32]], axis=1)
    z = jnp.concatenate([l_ref[...], right], axis=0)
    o_ref[...] = z.T


_pack_tc = pl.pallas_call(
    _pack_body,
    grid=(PGRID,),
    in_specs=[
        pl.BlockSpec((D, PKB), lambda b: (0, b)),
        pl.BlockSpec((D, PKB), lambda b: (0, RSHIFT + b)),
        pl.BlockSpec((D, PKB), lambda b: (0, jnp.minimum(RSHIFT + 1 + b, 2 * RSHIFT))),
    ],
    out_specs=pl.BlockSpec((PKB, PKB), lambda b: (b, 0)),
    out_shape=jax.ShapeDtypeStruct((HV, 2 * D), jnp.float32),
)


@functools.partial(
    pl.kernel,
    mesh=_mesh,
    compiler_params=pltpu.CompilerParams(use_tc_tiling_on_sc=False),
    out_type=jax.ShapeDtypeStruct((BS // 2, 2 * D), jnp.float32),
    scratch_types=[
        pltpu.VMEM((PER_W,), jnp.int32),
        pltpu.VMEM((PER_W,), jnp.int32),
        pltpu.VMEM((PER_W,), jnp.float32),
        pltpu.VMEM((ROWS_W,), jnp.float32),
        pltpu.VMEM((CHUNK, 2 * D), jnp.float32),
        pltpu.VMEM((OUT_SLAB // 2, 2 * D), jnp.float32),
        pltpu.SemaphoreType.DMA,
    ],
)
def _gaz_embed_sc(idx_hbm, mask_hbm, len_hbm, tbl_hbm, out_hbm,
                  idx_v, col_v, mask_v, len_v, rows_v, out_v, sem):
    wid = lax.axis_index("s") * NC + lax.axis_index("c")
    ibase = wid * PER_W
    rbase = wid * ROWS_W
    pltpu.sync_copy(idx_hbm.at[pl.ds(ibase, PER_W)], idx_v)
    pltpu.sync_copy(mask_hbm.at[pl.ds(ibase, PER_W)], mask_v)
    pltpu.sync_copy(len_hbm.at[pl.ds(rbase, ROWS_W)], len_v)

    def prep_body(t, _):
        sl = pl.ds(t * LANES, LANES)
        v = idx_v[sl]
        ge = v >= HV
        idx_v[sl] = v - jnp.where(ge, HV, 0)
        col_v[sl] = jnp.where(ge, D, 0)
        return 0

    lax.fori_loop(0, PER_W // LANES, prep_body, 0)

    def slab_body(s_i, _):
        soff = s_i * SLAB

        def chunk_body(c_i, _):
            coff = soff + c_i * CHUNK
            pltpu.async_copy(
                tbl_hbm.at[idx_v.at[pl.ds(coff, CHUNK)]], rows_v, sem
            ).wait()
            obase = c_i * (CHUNK // G)
            inv_vec = 1.0 / len_v[pl.ds(s_i * OUT_SLAB + obase, LANES)]
            for half in range(CHUNK // LANES):
                mv = mask_v[pl.ds(coff + half * LANES, LANES)]
                cv = col_v[pl.ds(coff + half * LANES, LANES)]
                for sub in range(2):
                    r = half * 2 + sub
                    r0 = r * G
                    inv = inv_vec[r]
                    opack = c_i * (CHUNK // G // 2) + r // 2
                    ocol = (r % 2) * D
                    for d_blk in range(D // LANES):
                        dof = d_blk * LANES
                        acc = rows_v[r0, pl.ds(cv[sub * G] + dof, LANES)] * mv[sub * G]
                        for g in range(1, G):
                            acc = acc + rows_v[r0 + g, pl.ds(cv[sub * G + g] + dof, LANES)] * mv[sub * G + g]
                        out_v[opack, pl.ds(ocol + dof, LANES)] = acc * inv
            return 0

        lax.fori_loop(0, CH_PER_SLAB, chunk_body, 0)
        pltpu.sync_copy(
            out_v,
            out_hbm.at[pl.ds((rbase + s_i * OUT_SLAB) // 2, OUT_SLAB // 2)],
        )
        return 0

    lax.fori_loop(0, NSLAB, slab_body, 0)


def kernel(gaz_seq_tensor, gaz_seq_lengths, gaz_mask_tensor, gaz_embedding):
    idx = gaz_seq_tensor.reshape(N).astype(jnp.int32)
    mask = gaz_mask_tensor.reshape(N)
    lens = gaz_seq_lengths.reshape(BS).astype(jnp.float32)
    tbl_t = gaz_embedding.T
    tbl2 = _pack_tc(tbl_t, tbl_t, tbl_t)
    out = _gaz_embed_sc(idx, mask, lens, tbl2)
    return out.reshape(B, S, D)


# speedup vs baseline: 3.2528x; 3.2528x over previous
import functools

import jax
import jax.numpy as jnp
from jax import lax
from jax.experimental import pallas as pl
from jax.experimental.pallas import tpu as pltpu
from jax.experimental.pallas import tpu_sc as plsc

B, S, G = 1024, 50, 8
D = 64
VOCAB = 1000000
N = B * S * G
BS = B * S
NC, NS = 2, 16
NW = NC * NS
PER_W = N // NW
ROWS_W = BS // NW
CHUNK = 128
SLAB = 1280
NSLAB = PER_W // SLAB
CH_PER_SLAB = SLAB // CHUNK
OUT_SLAB = SLAB // G
LANES = 16

_mesh = plsc.VectorSubcoreMesh(core_axis_name="c", subcore_axis_name="s")

HV = VOCAB // 2
PKB = 1024
PGRID = (HV + PKB - 1) // PKB
RSHIFT = HV // PKB
RCUT = HV - RSHIFT * PKB
RMAXB = (VOCAB - 1) // PKB


def _pack_body(l_ref, ra_ref, rb_ref, o_ref):
    right = jnp.concatenate(
        [ra_ref[:, RCUT:PKB], rb_ref[:, 0:RCUT]], axis=1)
    for t in range(PKB // 128):
        sl = pl.ds(t * 128, 128)
        z = jnp.concatenate([l_ref[:, sl], right[:, t * 128:(t + 1) * 128]],
                            axis=0)
        o_ref[sl, :] = z.T


_pack_tc = pl.pallas_call(
    _pack_body,
    grid=(PGRID,),
    in_specs=[
        pl.BlockSpec((D, PKB), lambda b: (0, b)),
        pl.BlockSpec((D, PKB), lambda b: (0, jnp.minimum(RSHIFT + b, RMAXB))),
        pl.BlockSpec((D, PKB), lambda b: (0, jnp.minimum(RSHIFT + 1 + b, RMAXB))),
    ],
    out_specs=pl.BlockSpec((PKB, 2 * D), lambda b: (b, 0)),
    out_shape=jax.ShapeDtypeStruct((HV, 2 * D), jnp.float32),
)


@functools.partial(
    pl.kernel,
    mesh=_mesh,
    compiler_params=pltpu.CompilerParams(use_tc_tiling_on_sc=False),
    out_type=jax.ShapeDtypeStruct((BS // 2, 2 * D), jnp.float32),
    scratch_types=[
        pltpu.VMEM((PER_W,), jnp.int32),
        pltpu.VMEM((PER_W,), jnp.int32),
        pltpu.VMEM((PER_W,), jnp.float32),
        pltpu.VMEM((ROWS_W,), jnp.float32),
        pltpu.VMEM((CHUNK, 2 * D), jnp.float32),
        pltpu.VMEM((OUT_SLAB // 2, 2 * D), jnp.float32),
        pltpu.SemaphoreType.DMA,
    ],
)
def _gaz_embed_sc(idx_hbm, mask_hbm, len_hbm, tbl_hbm, out_hbm,
                  idx_v, col_v, mask_v, len_v, rows_v, out_v, sem):
    wid = lax.axis_index("s") * NC + lax.axis_index("c")
    ibase = wid * PER_W
    rbase = wid * ROWS_W
    pltpu.sync_copy(idx_hbm.at[pl.ds(ibase, PER_W)], idx_v)
    pltpu.sync_copy(mask_hbm.at[pl.ds(ibase, PER_W)], mask_v)
    pltpu.sync_copy(len_hbm.at[pl.ds(rbase, ROWS_W)], len_v)

    def prep_body(t, _):
        sl = pl.ds(t * LANES, LANES)
        v = idx_v[sl]
        ge = v >= HV
        idx_v[sl] = v - jnp.where(ge, HV, 0)
        col_v[sl] = jnp.where(ge, D, 0)
        return 0

    lax.fori_loop(0, PER_W // LANES, prep_body, 0)

    def slab_body(s_i, _):
        soff = s_i * SLAB

        def chunk_body(c_i, _):
            coff = soff + c_i * CHUNK
            pltpu.async_copy(
                tbl_hbm.at[idx_v.at[pl.ds(coff, CHUNK)]], rows_v, sem
            ).wait()
            obase = c_i * (CHUNK // G)
            inv_vec = 1.0 / len_v[pl.ds(s_i * OUT_SLAB + obase, LANES)]
            for half in range(CHUNK // LANES):
                mv = mask_v[pl.ds(coff + half * LANES, LANES)]
                cv = col_v[pl.ds(coff + half * LANES, LANES)]
                for sub in range(2):
                    r = half * 2 + sub
                    r0 = r * G
                    inv = inv_vec[r]
                    opack = c_i * (CHUNK // G // 2) + r // 2
                    ocol = (r % 2) * D
                    for d_blk in range(D // LANES):
                        dof = d_blk * LANES
                        acc = rows_v[r0, pl.ds(cv[sub * G] + dof, LANES)] * mv[sub * G]
                        for g in range(1, G):
                            acc = acc + rows_v[r0 + g, pl.ds(cv[sub * G + g] + dof, LANES)] * mv[sub * G + g]
                        out_v[opack, pl.ds(ocol + dof, LANES)] = acc * inv
            return 0

        lax.fori_loop(0, CH_PER_SLAB, chunk_body, 0)
        pltpu.sync_copy(
            out_v,
            out_hbm.at[pl.ds((rbase + s_i * OUT_SLAB) // 2, OUT_SLAB // 2)],
        )
        return 0

    lax.fori_loop(0, NSLAB, slab_body, 0)


def kernel(gaz_seq_tensor, gaz_seq_lengths, gaz_mask_tensor, gaz_embedding):
    idx = gaz_seq_tensor.reshape(N).astype(jnp.int32)
    mask = gaz_mask_tensor.reshape(N)
    lens = gaz_seq_lengths.reshape(BS).astype(jnp.float32)
    tbl_t = gaz_embedding.T
    tbl2 = _pack_tc(tbl_t, tbl_t, tbl_t)
    out = _gaz_embed_sc(idx, mask, lens, tbl2)
    return out.reshape(B, S, D)


# speedup vs baseline: 4.3173x; 1.3273x over previous
import functools

import jax
import jax.numpy as jnp
from jax import lax
from jax.experimental import pallas as pl
from jax.experimental.pallas import tpu as pltpu
from jax.experimental.pallas import tpu_sc as plsc

B, S, G = 1024, 50, 8
D = 64
VOCAB = 1000000
N = B * S * G
BS = B * S
NC, NS = 2, 16
NW = NC * NS
PER_W = N // NW
ROWS_W = BS // NW
CHUNK = 128
SLAB = 1280
NSLAB = PER_W // SLAB
CH_PER_SLAB = SLAB // CHUNK
OUT_SLAB = SLAB // G
LANES = 16

_mesh = plsc.VectorSubcoreMesh(core_axis_name="c", subcore_axis_name="s")

HV = VOCAB // 2
PKB = 4096
PGRID = (HV + PKB - 1) // PKB
RSHIFT = HV // PKB
RCUT = HV - RSHIFT * PKB
RMAXB = (VOCAB - 1) // PKB


def _pack_body(l_ref, ra_ref, rb_ref, o_ref):
    right = jnp.concatenate(
        [ra_ref[:, RCUT:PKB], rb_ref[:, 0:RCUT]], axis=1)
    for t in range(PKB // 128):
        sl = pl.ds(t * 128, 128)
        z = jnp.concatenate([l_ref[:, sl], right[:, t * 128:(t + 1) * 128]],
                            axis=0)
        o_ref[sl, :] = z.T


_pack_tc = pl.pallas_call(
    _pack_body,
    grid=(PGRID,),
    in_specs=[
        pl.BlockSpec((D, PKB), lambda b: (0, b)),
        pl.BlockSpec((D, PKB), lambda b: (0, jnp.minimum(RSHIFT + b, RMAXB))),
        pl.BlockSpec((D, PKB), lambda b: (0, jnp.minimum(RSHIFT + 1 + b, RMAXB))),
    ],
    out_specs=pl.BlockSpec((PKB, 2 * D), lambda b: (b, 0)),
    out_shape=jax.ShapeDtypeStruct((HV, 2 * D), jnp.float32),
)


@functools.partial(
    pl.kernel,
    mesh=_mesh,
    compiler_params=pltpu.CompilerParams(use_tc_tiling_on_sc=False),
    out_type=jax.ShapeDtypeStruct((BS // 2, 2 * D), jnp.float32),
    scratch_types=[
        pltpu.VMEM((PER_W,), jnp.int32),
        pltpu.VMEM((PER_W,), jnp.int32),
        pltpu.VMEM((PER_W,), jnp.float32),
        pltpu.VMEM((ROWS_W,), jnp.float32),
        pltpu.VMEM((CHUNK, 2 * D), jnp.float32),
        pltpu.VMEM((OUT_SLAB // 2, 2 * D), jnp.float32),
        pltpu.SemaphoreType.DMA,
    ],
)
def _gaz_embed_sc(idx_hbm, mask_hbm, len_hbm, tbl_hbm, out_hbm,
                  idx_v, col_v, mask_v, len_v, rows_v, out_v, sem):
    wid = lax.axis_index("s") * NC + lax.axis_index("c")
    ibase = wid * PER_W
    rbase = wid * ROWS_W
    pltpu.sync_copy(idx_hbm.at[pl.ds(ibase, PER_W)], idx_v)
    pltpu.sync_copy(mask_hbm.at[pl.ds(ibase, PER_W)], mask_v)
    pltpu.sync_copy(len_hbm.at[pl.ds(rbase, ROWS_W)], len_v)

    def prep_body(t, _):
        sl = pl.ds(t * LANES, LANES)
        v = idx_v[sl]
        ge = v >= HV
        idx_v[sl] = v - jnp.where(ge, HV, 0)
        col_v[sl] = jnp.where(ge, D, 0)
        return 0

    lax.fori_loop(0, PER_W // LANES, prep_body, 0)

    def slab_body(s_i, _):
        soff = s_i * SLAB

        def chunk_body(c_i, _):
            coff = soff + c_i * CHUNK
            pltpu.async_copy(
                tbl_hbm.at[idx_v.at[pl.ds(coff, CHUNK)]], rows_v, sem
            ).wait()
            obase = c_i * (CHUNK // G)
            inv_vec = 1.0 / len_v[pl.ds(s_i * OUT_SLAB + obase, LANES)]
            for half in range(CHUNK // LANES):
                mv = mask_v[pl.ds(coff + half * LANES, LANES)]
                cv = col_v[pl.ds(coff + half * LANES, LANES)]
                for sub in range(2):
                    r = half * 2 + sub
                    r0 = r * G
                    inv = inv_vec[r]
                    opack = c_i * (CHUNK // G // 2) + r // 2
                    ocol = (r % 2) * D
                    for d_blk in range(D // LANES):
                        dof = d_blk * LANES
                        acc = rows_v[r0, pl.ds(cv[sub * G] + dof, LANES)] * mv[sub * G]
                        for g in range(1, G):
                            acc = acc + rows_v[r0 + g, pl.ds(cv[sub * G + g] + dof, LANES)] * mv[sub * G + g]
                        out_v[opack, pl.ds(ocol + dof, LANES)] = acc * inv
            return 0

        lax.fori_loop(0, CH_PER_SLAB, chunk_body, 0)
        pltpu.sync_copy(
            out_v,
            out_hbm.at[pl.ds((rbase + s_i * OUT_SLAB) // 2, OUT_SLAB // 2)],
        )
        return 0

    lax.fori_loop(0, NSLAB, slab_body, 0)


def kernel(gaz_seq_tensor, gaz_seq_lengths, gaz_mask_tensor, gaz_embedding):
    idx = gaz_seq_tensor.reshape(N).astype(jnp.int32)
    mask = gaz_mask_tensor.reshape(N)
    lens = gaz_seq_lengths.reshape(BS).astype(jnp.float32)
    tbl_t = gaz_embedding.T
    tbl2 = _pack_tc(tbl_t, tbl_t, tbl_t)
    out = _gaz_embed_sc(idx, mask, lens, tbl2)
    return out.reshape(B, S, D)


# speedup vs baseline: 4.7053x; 1.0899x over previous
import functools

import jax
import jax.numpy as jnp
from jax import lax
from jax.experimental import pallas as pl
from jax.experimental.pallas import tpu as pltpu
from jax.experimental.pallas import tpu_sc as plsc

B, S, G = 1024, 50, 8
D = 64
VOCAB = 1000000
N = B * S * G
BS = B * S
NC, NS = 2, 16
NW = NC * NS
PER_W = N // NW
ROWS_W = BS // NW
CHUNK = 128
NBUF = 4
SLAB = NBUF * CHUNK
NSLAB = PER_W // SLAB
OUT_SLAB = SLAB // G
LANES = 16

PKB = 4096
PHV = 123 * PKB
PGRID = PHV // PKB
RMAXB = (VOCAB - 1) // PKB

_mesh = plsc.VectorSubcoreMesh(core_axis_name="c", subcore_axis_name="s")


def _pack_body(l_ref, r_ref, o_ref):
    for t in range(PKB // 128):
        sl = pl.ds(t * 128, 128)
        z = jnp.concatenate([l_ref[:, sl], r_ref[:, sl]], axis=0)
        o_ref[sl, :] = z.T


_pack_tc = pl.pallas_call(
    _pack_body,
    grid=(PGRID,),
    in_specs=[
        pl.BlockSpec((D, PKB), lambda b: (0, b)),
        pl.BlockSpec((D, PKB), lambda b: (0, jnp.minimum(PGRID + b, RMAXB))),
    ],
    out_specs=pl.BlockSpec((PKB, 2 * D), lambda b: (b, 0)),
    out_shape=jax.ShapeDtypeStruct((PHV, 2 * D), jnp.float32),
)


@functools.partial(
    pl.kernel,
    mesh=_mesh,
    compiler_params=pltpu.CompilerParams(use_tc_tiling_on_sc=False),
    out_type=jax.ShapeDtypeStruct((BS // 2, 2 * D), jnp.float32),
    scratch_types=[
        pltpu.VMEM((PER_W,), jnp.int32),
        pltpu.VMEM((PER_W,), jnp.int32),
        pltpu.VMEM((PER_W,), jnp.float32),
        pltpu.VMEM((ROWS_W,), jnp.float32),
        pltpu.VMEM((CHUNK, 2 * D), jnp.float32),
        pltpu.VMEM((CHUNK, 2 * D), jnp.float32),
        pltpu.VMEM((CHUNK, 2 * D), jnp.float32),
        pltpu.VMEM((CHUNK, 2 * D), jnp.float32),
        pltpu.VMEM((OUT_SLAB // 2, 2 * D), jnp.float32),
        pltpu.SemaphoreType.DMA,
        pltpu.SemaphoreType.DMA,
        pltpu.SemaphoreType.DMA,
        pltpu.SemaphoreType.DMA,
    ],
)
def _gaz_embed_sc(idx_hbm, mask_hbm, len_hbm, tbl_hbm, out_hbm,
                  idx_v, col_v, mask_v, len_v, rv0, rv1, rv2, rv3,
                  out_v, sem0, sem1, sem2, sem3):
    rows_bufs = (rv0, rv1, rv2, rv3)
    sems = (sem0, sem1, sem2, sem3)
    wid = lax.axis_index("s") * NC + lax.axis_index("c")
    ibase = wid * PER_W
    rbase = wid * ROWS_W
    pltpu.sync_copy(idx_hbm.at[pl.ds(ibase, PER_W)], idx_v)
    pltpu.sync_copy(mask_hbm.at[pl.ds(ibase, PER_W)], mask_v)
    pltpu.sync_copy(len_hbm.at[pl.ds(rbase, ROWS_W)], len_v)

    def prep_body(t, _):
        sl = pl.ds(t * LANES, LANES)
        v = idx_v[sl]
        ge = v >= PHV
        idx_v[sl] = v - jnp.where(ge, PHV, 0)
        col_v[sl] = jnp.where(ge, D, 0)
        return 0

    lax.fori_loop(0, PER_W // LANES, prep_body, 0)

    def gather(chunk_off, buf, sem):
        return pltpu.async_copy(
            tbl_hbm.at[idx_v.at[pl.ds(chunk_off, CHUNK)]], buf, sem)

    for b in range(NBUF):
        gather(b * CHUNK, rows_bufs[b], sems[b])

    def slab_body(s_i, _):
        soff = s_i * SLAB
        for b in range(NBUF):
            coff = soff + b * CHUNK
            rows_v = rows_bufs[b]
            pltpu.make_async_copy(
                tbl_hbm.at[idx_v.at[pl.ds(coff, CHUNK)]], rows_v, sems[b]
            ).wait()
            obase = b * (CHUNK // G)
            inv_vec = 1.0 / len_v[pl.ds(s_i * OUT_SLAB + obase, LANES)]
            for half in range(CHUNK // LANES):
                mv = mask_v[pl.ds(coff + half * LANES, LANES)]
                cv = col_v[pl.ds(coff + half * LANES, LANES)]
                for sub in range(2):
                    r = half * 2 + sub
                    r0 = r * G
                    inv = inv_vec[r]
                    opack = (obase + r) // 2
                    ocol = (r % 2) * D
                    for d_blk in range(D // LANES):
                        dof = d_blk * LANES
                        acc = rows_v[r0, pl.ds(cv[sub * G] + dof, LANES)] * mv[sub * G]
                        for g in range(1, G):
                            acc = acc + rows_v[r0 + g, pl.ds(cv[sub * G + g] + dof, LANES)] * mv[sub * G + g]
                        out_v[opack, pl.ds(ocol + dof, LANES)] = acc * inv
            @pl.when(s_i < NSLAB - 1)
            def _():
                gather(coff + SLAB, rows_v, sems[b])

        pltpu.sync_copy(
            out_v,
            out_hbm.at[pl.ds((rbase + s_i * OUT_SLAB) // 2, OUT_SLAB // 2)],
        )
        return 0

    lax.fori_loop(0, NSLAB, slab_body, 0)


def kernel(gaz_seq_tensor, gaz_seq_lengths, gaz_mask_tensor, gaz_embedding):
    idx = gaz_seq_tensor.reshape(N).astype(jnp.int32)
    mask = gaz_mask_tensor.reshape(N)
    lens = gaz_seq_lengths.reshape(BS).astype(jnp.float32)
    tbl_t = gaz_embedding.T
    tbl2 = _pack_tc(tbl_t, tbl_t)
    out = _gaz_embed_sc(idx, mask, lens, tbl2)
    return out.reshape(B, S, D)
